# baseline (speedup 1.0000x reference)
import functools
import math

import jax
import jax.numpy as jnp
from jax import lax
from jax.experimental import pallas as pl
from jax.experimental.pallas import tpu as pltpu
from jax.experimental.pallas import tpu_sc as plsc

VOCAB = 1000000
D_MODEL = 64
BATCH = 4096
HIST = 200

NUM_CORES = 2
NUM_SUBCORES = 16
NW = NUM_CORES * NUM_SUBCORES

LANES = 16
WPAD = 128
BBLK = BATCH // NW
HC = HIST // 8
SCALE = math.sqrt(D_MODEL)

VCHUNK = 128
NCHUNK = (VOCAB + VCHUNK - 1) // VCHUNK
VPAD = NCHUNK * VCHUNK
TAIL = NCHUNK - 1
CPW = (NCHUNK + NW - 1) // NW

_mesh = plsc.VectorSubcoreMesh(core_axis_name="c", subcore_axis_name="s")


@functools.partial(
    pl.kernel,
    out_type=jax.ShapeDtypeStruct((VPAD, WPAD), jnp.float32),
    mesh=_mesh,
    scratch_types=[
        pltpu.VMEM((D_MODEL, VCHUNK), jnp.float32),
        pltpu.VMEM((D_MODEL, VCHUNK), jnp.float32),
        pltpu.VMEM((VCHUNK, 129), jnp.float32),
        pltpu.VMEM((VCHUNK, 129), jnp.float32),
        pltpu.SemaphoreType.DMA,
        pltpu.SemaphoreType.DMA,
        pltpu.SemaphoreType.DMA,
        pltpu.SemaphoreType.DMA,
    ],
    compiler_params=pltpu.CompilerParams(
        use_tc_tiling_on_sc=True, needs_layout_passes=False),
)
def _table_rows(tt_hbm, pt_hbm, a0, a1, t0, t1, ai0, ai1, to0, to1):
    wid = lax.axis_index("s") * NUM_CORES + lax.axis_index("c")
    abufs, tbufs = (a0, a1), (t0, t1)
    isems, osems = (ai0, ai1), (to0, to1)
    base = wid * CPW

    def chunk_of(k):
        return lax.min(base + k, TAIL)

    def in_desc(k, b):
        return pltpu.make_async_copy(
            tt_hbm.at[:, pl.ds(chunk_of(k) * VCHUNK, VCHUNK)],
            abufs[b], isems[b])

    def out_desc(k, b):
        return pltpu.make_async_copy(
            tbufs[b].at[:, pl.ds(0, WPAD)],
            pt_hbm.at[pl.ds(chunk_of(k) * VCHUNK, VCHUNK)], osems[b])

    iota = lax.iota(jnp.int32, LANES)
    v_ids = [iota + vl * LANES for vl in range(VCHUNK // LANES)]
    zero_v = jnp.full((LANES,), 0, jnp.int32)

    in_desc(0, 0).start()
    in_desc(1, 1).start()

    @pl.loop(0, CPW + 1, step=2)
    def _visit(k0):
        for b in range(2):
            k = k0 + b
            in_desc(k, b).wait()

            @pl.when(k >= 2)
            def _():
                out_desc(k - 2, b).wait()

            @plsc.parallel_loop(0, D_MODEL, unroll=2)
            def _(c):
                cvec = zero_v + c
                for vl in range(VCHUNK // LANES):
                    v = abufs[b][c, pl.ds(vl * LANES, LANES)] * SCALE
                    plsc.store_scatter(tbufs[b], [v_ids[vl], cvec], v)

            @pl.when(k + 2 <= CPW)
            def _():
                in_desc(k + 2, b).start()

            out_desc(k, b).start()

    out_desc(CPW - 1, 0).wait()
    out_desc(CPW, 1).wait()


@functools.partial(
    pl.kernel,
    out_type=jax.ShapeDtypeStruct((HIST, 8, NW, 8, BBLK), jnp.float32),
    mesh=_mesh,
    scratch_types=[
        pltpu.VMEM((HC, 8, BBLK), jnp.int32),
        pltpu.VMEM((BBLK, WPAD), jnp.float32),
        pltpu.VMEM((BBLK, WPAD), jnp.float32),
        pltpu.VMEM((8, 8, BBLK + 1), jnp.float32),
        pltpu.VMEM((8, 8, BBLK + 1), jnp.float32),
        pltpu.SemaphoreType.DMA,
        pltpu.SemaphoreType.DMA,
        pltpu.SemaphoreType.DMA,
        pltpu.SemaphoreType.DMA,
    ],
    compiler_params=pltpu.CompilerParams(
        use_tc_tiling_on_sc=False, needs_layout_passes=False),
)
def _emb_lookup(xq_hbm, tp_hbm, out_hbm,
                idx_v, g0, g1, s0, s1, gs0, gs1, os0, os1):
    wid = lax.axis_index("s") * NUM_CORES + lax.axis_index("c")
    gbufs, sbufs = (g0, g1), (s0, s1)
    gsems, osems = (gs0, gs1), (os0, os1)

    pltpu.sync_copy(xq_hbm.at[:, wid], idx_v)

    def gather_desc(h, b):
        hc = lax.shift_right_logical(h, 3)
        hl = lax.bitwise_and(h, 7)
        return pltpu.make_async_copy(
            tp_hbm.at[idx_v.at[hc, hl]], gbufs[b], gsems[b])

    def out_desc(h, b):
        return pltpu.make_async_copy(
            sbufs[b].at[:, :, pl.ds(0, BBLK)], out_hbm.at[h, :, wid], osems[b])

    iota = lax.iota(jnp.int32, LANES)
    tr_ids = [lax.shift_right_logical(iota + jd * LANES, 3) for jd in range(4)]
    dlo_ids = [lax.bitwise_and(iota + jd * LANES, 7) for jd in range(4)]
    zero_v = jnp.full((LANES,), 0, jnp.int32)

    gather_desc(0, 0).start()
    gather_desc(1, 1).start()

    @pl.loop(0, HIST, step=2)
    def _visit(h0):
        for b in range(2):
            h = h0 + b
            gather_desc(h, b).wait()

            @pl.when(h >= 2)
            def _():
                out_desc(h, b).wait()

            @plsc.parallel_loop(0, BBLK, unroll=2)
            def _(r):
                blo = zero_v + r
                for jd in range(4):
                    v = gbufs[b][r, pl.ds(jd * LANES, LANES)]
                    plsc.store_scatter(
                        sbufs[b], [tr_ids[jd], dlo_ids[jd], blo], v)

            @pl.when(h + 2 < HIST)
            def _():
                gather_desc(h + 2, b).start()

            out_desc(h, b).start()

    for b in range(2):
        out_desc(HIST - 2 + b, b).wait()


def kernel(x, table):
    xq = x.T.reshape(HC, 8, NW, BBLK).transpose(0, 2, 1, 3)
    pt = _table_rows(table.T)
    lin = _emb_lookup(xq, pt)
    return lin.transpose(2, 4, 0, 1, 3).reshape(BATCH, HIST, D_MODEL)


# speedup vs baseline: 1.4426x; 1.4426x over previous
import functools
import math

import jax
import jax.numpy as jnp
from jax import lax
from jax.experimental import pallas as pl
from jax.experimental.pallas import tpu as pltpu
from jax.experimental.pallas import tpu_sc as plsc

VOCAB = 1000000
D_MODEL = 64
BATCH = 4096
HIST = 200

NUM_CORES = 2
NUM_SUBCORES = 16
NW = NUM_CORES * NUM_SUBCORES

LANES = 16
WPAD = 128
BBLK = BATCH // NW
HC = HIST // 8
SCALE = math.sqrt(D_MODEL)

_mesh = plsc.VectorSubcoreMesh(core_axis_name="c", subcore_axis_name="s")


@functools.partial(
    pl.kernel,
    out_type=jax.ShapeDtypeStruct((HIST, 8, NW, 8, BBLK), jnp.float32),
    mesh=_mesh,
    scratch_types=[
        pltpu.VMEM((HC, 8, BBLK), jnp.int32),
        pltpu.VMEM((BBLK, WPAD), jnp.float32),
        pltpu.VMEM((BBLK, WPAD), jnp.float32),
        pltpu.VMEM((8, 8, BBLK + 1), jnp.float32),
        pltpu.VMEM((8, 8, BBLK + 1), jnp.float32),
        pltpu.SemaphoreType.DMA,
        pltpu.SemaphoreType.DMA,
        pltpu.SemaphoreType.DMA,
        pltpu.SemaphoreType.DMA,
    ],
    compiler_params=pltpu.CompilerParams(
        use_tc_tiling_on_sc=False, needs_layout_passes=False),
)
def _emb_lookup(xq_hbm, tp_hbm, out_hbm,
                idx_v, g0, g1, s0, s1, gs0, gs1, os0, os1):
    wid = lax.axis_index("s") * NUM_CORES + lax.axis_index("c")
    gbufs, sbufs = (g0, g1), (s0, s1)
    gsems, osems = (gs0, gs1), (os0, os1)

    pltpu.sync_copy(xq_hbm.at[:, wid], idx_v)

    def gather_desc(h, b):
        hc = lax.shift_right_logical(h, 3)
        hl = lax.bitwise_and(h, 7)
        return pltpu.make_async_copy(
            tp_hbm.at[idx_v.at[hc, hl]], gbufs[b], gsems[b])

    def out_desc(h, b):
        return pltpu.make_async_copy(
            sbufs[b].at[:, :, pl.ds(0, BBLK)], out_hbm.at[h, :, wid], osems[b])

    iota = lax.iota(jnp.int32, LANES)
    tr_ids = [lax.shift_right_logical(iota + jd * LANES, 3) for jd in range(4)]
    dlo_ids = [lax.bitwise_and(iota + jd * LANES, 7) for jd in range(4)]
    zero_v = jnp.full((LANES,), 0, jnp.int32)

    gather_desc(0, 0).start()
    gather_desc(1, 1).start()

    @pl.loop(0, HIST, step=2)
    def _visit(h0):
        for b in range(2):
            h = h0 + b
            gather_desc(h, b).wait()

            @pl.when(h >= 2)
            def _():
                out_desc(h, b).wait()

            @plsc.parallel_loop(0, BBLK, unroll=2)
            def _(r):
                blo = zero_v + r
                for jd in range(4):
                    v = gbufs[b][r, pl.ds(jd * LANES, LANES)] * SCALE
                    plsc.store_scatter(
                        sbufs[b], [tr_ids[jd], dlo_ids[jd], blo], v)

            @pl.when(h + 2 < HIST)
            def _():
                gather_desc(h + 2, b).start()

            out_desc(h, b).start()

    for b in range(2):
        out_desc(HIST - 2 + b, b).wait()


def kernel(x, table):
    xq = x.T.reshape(HC, 8, NW, BBLK).transpose(0, 2, 1, 3)
    tp = jnp.pad(table, ((0, 0), (0, WPAD - D_MODEL)))
    lin = _emb_lookup(xq, tp)
    return lin.transpose(2, 4, 0, 1, 3).reshape(BATCH, HIST, D_MODEL)
